# baseline (speedup 1.0000x reference)
import functools

import jax
import jax.numpy as jnp
from jax import lax
from jax.experimental import pallas as pl
from jax.experimental.pallas import tpu as pltpu
from jax.experimental.pallas import tpu_sc as plsc

B = 2048
V = 512
FEAT_DIM = 16
D_MODEL = 64
NUM_SAMPLES = 256

N = B * V
GATHER_W = 128
NUM_IDX_ROWS = N // GATHER_W
MASK_BIG = 1e4


def _make_sc_gather():
    info = plsc.get_sparse_core_info()
    nc, ns = info.num_cores, info.num_subcores
    nw = nc * ns
    rows_per_w = NUM_IDX_ROWS // nw

    mesh = plsc.VectorSubcoreMesh(core_axis_name="c", subcore_axis_name="s")

    @functools.partial(
        pl.kernel,
        mesh=mesh,
        compiler_params=pltpu.CompilerParams(use_tc_tiling_on_sc=False),
        out_type=jax.ShapeDtypeStruct((N, D_MODEL), jnp.float32),
        scratch_types=[
            pltpu.VMEM((2 * GATHER_W,), jnp.int32),
            pltpu.VMEM((GATHER_W, D_MODEL), jnp.float32),
            pltpu.VMEM((GATHER_W, D_MODEL), jnp.float32),
            pltpu.SemaphoreType.DMA,
            pltpu.SemaphoreType.DMA,
            pltpu.SemaphoreType.DMA,
            pltpu.SemaphoreType.DMA,
        ],
    )
    def sc_gather(idx_hbm, table_hbm, out_hbm, idx2, rows0, rows1,
                  gsem0, gsem1, wsem0, wsem1):
        wid = lax.axis_index("s") * nc + lax.axis_index("c")
        base = wid * rows_per_w

        def body(g, carry):
            r0 = base + 2 * g
            pltpu.sync_copy(idx_hbm.at[pl.ds(r0 * GATHER_W, 2 * GATHER_W)],
                            idx2)

            @pl.when(g > 0)
            def _w0():
                pltpu.make_async_copy(
                    rows0, out_hbm.at[pl.ds((r0 - 2) * GATHER_W, GATHER_W)],
                    wsem0).wait()

            pltpu.async_copy(table_hbm.at[idx2.at[pl.ds(0, GATHER_W)]],
                             rows0, gsem0)

            @pl.when(g > 0)
            def _w1():
                pltpu.make_async_copy(
                    rows1, out_hbm.at[pl.ds((r0 - 1) * GATHER_W, GATHER_W)],
                    wsem1).wait()

            pltpu.async_copy(table_hbm.at[idx2.at[pl.ds(GATHER_W, GATHER_W)]],
                             rows1, gsem1)

            pltpu.make_async_copy(table_hbm.at[idx2.at[pl.ds(0, GATHER_W)]],
                                  rows0, gsem0).wait()
            pltpu.async_copy(rows0,
                             out_hbm.at[pl.ds(r0 * GATHER_W, GATHER_W)],
                             wsem0)
            pltpu.make_async_copy(table_hbm.at[idx2.at[pl.ds(GATHER_W,
                                                             GATHER_W)]],
                                  rows1, gsem1).wait()
            pltpu.async_copy(rows1,
                             out_hbm.at[pl.ds((r0 + 1) * GATHER_W, GATHER_W)],
                             wsem1)
            return carry

        lax.fori_loop(0, rows_per_w // 2, body, 0)
        last = base + rows_per_w - 2
        pltpu.make_async_copy(
            rows0, out_hbm.at[pl.ds(last * GATHER_W, GATHER_W)], wsem0).wait()
        pltpu.make_async_copy(
            rows1, out_hbm.at[pl.ds((last + 1) * GATHER_W, GATHER_W)],
            wsem1).wait()

    return sc_gather


_NB = 64
_PK = 8
_RW = _NB * V // _PK
_LW = _PK * D_MODEL


def _tanh(x):
    x = jnp.clip(x, -7.90531110763549805, 7.90531110763549805)
    x2 = x * x
    p = jnp.float32(-2.76076847742355e-16)
    p = p * x2 + jnp.float32(2.00018790482477e-13)
    p = p * x2 + jnp.float32(-8.60467152213735e-11)
    p = p * x2 + jnp.float32(5.12229709037114e-08)
    p = p * x2 + jnp.float32(1.48572235717979e-05)
    p = p * x2 + jnp.float32(6.37261928875436e-04)
    p = p * x2 + jnp.float32(4.89352455891786e-03)
    p = p * x
    q = jnp.float32(1.19825839466702e-06)
    q = q * x2 + jnp.float32(1.18534705686654e-04)
    q = q * x2 + jnp.float32(2.26843463243900e-03)
    q = q * x2 + jnp.float32(4.89352518554385e-03)
    return p / q


def _tc_body(ge0_ref, ge1_ref, ge2_ref, ge3_ref, ft_ref, pm_ref, sid_ref,
             w8_ref, s8_ref, r2_ref, one_ref, wc_ref, out_ref, accs, accc):
    i = pl.program_id(0)

    @pl.when(i == 0)
    def _init():
        accs[...] = jnp.zeros_like(accs)
        accc[...] = jnp.zeros_like(accc)

    pmx = jnp.dot(pm_ref[...], s8_ref[...],
                  preferred_element_type=jnp.float32,
                  precision=jax.lax.Precision.HIGHEST)
    fp = jnp.dot(ft_ref[...], w8_ref[...],
                 preferred_element_type=jnp.float32)
    psum = jnp.zeros((_NB, 2 * D_MODEL), jnp.float32)
    dsum = jnp.zeros((_NB, 2 * D_MODEL), jnp.float32)
    for t, ge_ref in enumerate((ge0_ref, ge1_ref, ge2_ref, ge3_ref)):
        lo = t * 2 * D_MODEL
        pmx_t = pmx[:, lo:lo + 2 * D_MODEL]
        x = ge_ref[...] + fp[:, lo:lo + 2 * D_MODEL] + pmx_t
        h = jnp.tanh(x)
        mv = (pmx_t < MASK_BIG * 0.5).astype(jnp.float32)
        tm = h * mv
        psum = psum + tm.reshape(_NB, V // _PK, 2 * D_MODEL).sum(axis=1)
        dsum = dsum + mv.reshape(_NB, V // _PK, 2 * D_MODEL).sum(axis=1)
    pooled_sum = jnp.dot(psum, r2_ref[...],
                         preferred_element_type=jnp.float32,
                         precision=jax.lax.Precision.HIGHEST)
    den = jnp.dot(dsum, one_ref[...],
                  preferred_element_type=jnp.float32,
                  precision=jax.lax.Precision.HIGHEST) * (1.0 / D_MODEL)
    pooled = pooled_sum / jnp.maximum(den, 1.0)
    logit = jnp.dot(pooled, wc_ref[...],
                    preferred_element_type=jnp.float32)

    ids = sid_ref[...]
    lanes = lax.broadcasted_iota(jnp.int32, (_NB, NUM_SAMPLES), 1)
    oh = (ids == lanes).astype(jnp.float32)
    accs[...] += oh * logit
    accc[...] += oh

    @pl.when(i == pl.num_programs(0) - 1)
    def _fin():
        ssum = accs[...].sum(axis=0, keepdims=True)
        csum = accc[...].sum(axis=0, keepdims=True)
        out_ref[...] = ssum / jnp.maximum(csum, 1.0)


def _tc_call(ge2, ft8, pm8, sid, W8, S8, R2, ONE, wc):
    grid = B // _NB
    nsteps = (N // _PK) // _RW

    def _ge_spec(t):
        return pl.BlockSpec((_RW, 2 * D_MODEL),
                            lambda i, t=t: (t * nsteps + i, 0))

    return pl.pallas_call(
        _tc_body,
        grid=(grid,),
        in_specs=[
            _ge_spec(0), _ge_spec(1), _ge_spec(2), _ge_spec(3),
            pl.BlockSpec((_RW, _PK * FEAT_DIM), lambda i: (i, 0)),
            pl.BlockSpec((_RW, _PK), lambda i: (i, 0)),
            pl.BlockSpec((_NB, 1), lambda i: (i, 0)),
            pl.BlockSpec((_PK * FEAT_DIM, _LW), lambda i: (0, 0)),
            pl.BlockSpec((_PK, _LW), lambda i: (0, 0)),
            pl.BlockSpec((2 * D_MODEL, D_MODEL), lambda i: (0, 0)),
            pl.BlockSpec((2 * D_MODEL, 1), lambda i: (0, 0)),
            pl.BlockSpec((D_MODEL, 1), lambda i: (0, 0)),
        ],
        out_specs=pl.BlockSpec((1, NUM_SAMPLES), lambda i: (0, 0)),
        out_shape=jax.ShapeDtypeStruct((1, NUM_SAMPLES), jnp.float32),
        scratch_shapes=[
            pltpu.VMEM((_NB, NUM_SAMPLES), jnp.float32),
            pltpu.VMEM((_NB, NUM_SAMPLES), jnp.float32),
        ],
    )(ge2, ge2, ge2, ge2, ft8, pm8, sid, W8, S8, R2, ONE, wc)


def kernel(features, positions, gene_ids, mask, original_sample_indices,
           gene_table, W_feat, w_cls):
    ids_perm = (gene_ids.reshape(N).astype(jnp.int32)
                .reshape(N // _PK, 4, 2).transpose(1, 0, 2).reshape(N))
    ge = _make_sc_gather()(ids_perm, gene_table)

    ge2 = ge.reshape(N // 2, 2 * D_MODEL)
    ft8 = features.reshape(N // _PK, _PK * FEAT_DIM)
    pm8 = (positions.astype(jnp.float32) / 1e5
           + (1.0 - mask.astype(jnp.float32)) * MASK_BIG).reshape(N // _PK,
                                                                  _PK)
    sid = original_sample_indices.reshape(B, 1).astype(jnp.int32)
    wc = w_cls.reshape(D_MODEL, 1)

    eye8 = jnp.eye(_PK, dtype=jnp.float32)
    W8 = jnp.kron(eye8, W_feat)
    S8 = jnp.kron(eye8, jnp.ones((1, D_MODEL), jnp.float32))
    R2 = jnp.kron(jnp.ones((2, 1), jnp.float32),
                  jnp.eye(D_MODEL, dtype=jnp.float32))
    ONE = jnp.ones((2 * D_MODEL, 1), jnp.float32)

    out = _tc_call(ge2, ft8, pm8, sid, W8, S8, R2, ONE, wc)
    return out.reshape(NUM_SAMPLES)


# speedup vs baseline: 1.3450x; 1.3450x over previous
import functools

import jax
import jax.numpy as jnp
from jax import lax
from jax.experimental import pallas as pl
from jax.experimental.pallas import tpu as pltpu
from jax.experimental.pallas import tpu_sc as plsc

B = 2048
V = 512
FEAT_DIM = 16
D_MODEL = 64
NUM_SAMPLES = 256

N = B * V
GATHER_W = 128
NUM_IDX_ROWS = N // GATHER_W
MASK_BIG = 1e4


def _make_sc_gather():
    info = plsc.get_sparse_core_info()
    nc, ns = info.num_cores, info.num_subcores
    nw = nc * ns
    rows_per_w = NUM_IDX_ROWS // nw

    mesh = plsc.VectorSubcoreMesh(core_axis_name="c", subcore_axis_name="s")

    @functools.partial(
        pl.kernel,
        mesh=mesh,
        compiler_params=pltpu.CompilerParams(use_tc_tiling_on_sc=False),
        out_type=jax.ShapeDtypeStruct((N, D_MODEL), jnp.float32),
        scratch_types=[
            pltpu.VMEM((2 * GATHER_W,), jnp.int32),
            pltpu.VMEM((GATHER_W, D_MODEL), jnp.float32),
            pltpu.VMEM((GATHER_W, D_MODEL), jnp.float32),
            pltpu.SemaphoreType.DMA,
            pltpu.SemaphoreType.DMA,
            pltpu.SemaphoreType.DMA,
            pltpu.SemaphoreType.DMA,
        ],
    )
    def sc_gather(idx_hbm, table_hbm, out_hbm, idx2, rows0, rows1,
                  gsem0, gsem1, wsem0, wsem1):
        wid = lax.axis_index("s") * nc + lax.axis_index("c")
        base = wid * rows_per_w

        def body(g, carry):
            r0 = base + 2 * g
            pltpu.sync_copy(idx_hbm.at[pl.ds(r0 * GATHER_W, 2 * GATHER_W)],
                            idx2)

            @pl.when(g > 0)
            def _w0():
                pltpu.make_async_copy(
                    rows0, out_hbm.at[pl.ds((r0 - 2) * GATHER_W, GATHER_W)],
                    wsem0).wait()

            pltpu.async_copy(table_hbm.at[idx2.at[pl.ds(0, GATHER_W)]],
                             rows0, gsem0)

            @pl.when(g > 0)
            def _w1():
                pltpu.make_async_copy(
                    rows1, out_hbm.at[pl.ds((r0 - 1) * GATHER_W, GATHER_W)],
                    wsem1).wait()

            pltpu.async_copy(table_hbm.at[idx2.at[pl.ds(GATHER_W, GATHER_W)]],
                             rows1, gsem1)

            pltpu.make_async_copy(table_hbm.at[idx2.at[pl.ds(0, GATHER_W)]],
                                  rows0, gsem0).wait()
            pltpu.async_copy(rows0,
                             out_hbm.at[pl.ds(r0 * GATHER_W, GATHER_W)],
                             wsem0)
            pltpu.make_async_copy(table_hbm.at[idx2.at[pl.ds(GATHER_W,
                                                             GATHER_W)]],
                                  rows1, gsem1).wait()
            pltpu.async_copy(rows1,
                             out_hbm.at[pl.ds((r0 + 1) * GATHER_W, GATHER_W)],
                             wsem1)
            return carry

        lax.fori_loop(0, rows_per_w // 2, body, 0)
        last = base + rows_per_w - 2
        pltpu.make_async_copy(
            rows0, out_hbm.at[pl.ds(last * GATHER_W, GATHER_W)], wsem0).wait()
        pltpu.make_async_copy(
            rows1, out_hbm.at[pl.ds((last + 1) * GATHER_W, GATHER_W)],
            wsem1).wait()

    return sc_gather


_NB = 64
_PK = 8
_RW = _NB * V // _PK
_LW = _PK * D_MODEL


def _tanh(x):
    x = jnp.clip(x, -7.90531110763549805, 7.90531110763549805)
    x2 = x * x
    p = jnp.float32(-2.76076847742355e-16)
    p = p * x2 + jnp.float32(2.00018790482477e-13)
    p = p * x2 + jnp.float32(-8.60467152213735e-11)
    p = p * x2 + jnp.float32(5.12229709037114e-08)
    p = p * x2 + jnp.float32(1.48572235717979e-05)
    p = p * x2 + jnp.float32(6.37261928875436e-04)
    p = p * x2 + jnp.float32(4.89352455891786e-03)
    p = p * x
    q = jnp.float32(1.19825839466702e-06)
    q = q * x2 + jnp.float32(1.18534705686654e-04)
    q = q * x2 + jnp.float32(2.26843463243900e-03)
    q = q * x2 + jnp.float32(4.89352518554385e-03)
    return p / q


def _tc_body(ge_ref, ft_ref, pm_ref, sid_ref, w8_ref, s8_ref, r8_ref,
             one_ref, wc_ref, out_ref, accs, accc):
    i = pl.program_id(0)

    @pl.when(i == 0)
    def _init():
        accs[...] = jnp.zeros_like(accs)
        accc[...] = jnp.zeros_like(accc)

    pmx = jnp.dot(pm_ref[...], s8_ref[...],
                  preferred_element_type=jnp.float32,
                  precision=jax.lax.Precision.HIGHEST)
    fp = jnp.dot(ft_ref[...], w8_ref[...],
                 preferred_element_type=jnp.float32)
    x = ge_ref[...] + fp + pmx
    h = jnp.tanh(x)
    mvec = (pmx < MASK_BIG * 0.5).astype(jnp.float32)
    t = h * mvec
    psum = t.reshape(_NB, V // _PK, _LW).sum(axis=1)
    pooled_sum = jnp.dot(psum, r8_ref[...],
                         preferred_element_type=jnp.float32,
                         precision=jax.lax.Precision.HIGHEST)
    dsum = mvec.reshape(_NB, V // _PK, _LW).sum(axis=1)
    den = jnp.dot(dsum, one_ref[...],
                  preferred_element_type=jnp.float32,
                  precision=jax.lax.Precision.HIGHEST) * (1.0 / D_MODEL)
    pooled = pooled_sum / jnp.maximum(den, 1.0)
    logit = jnp.dot(pooled, wc_ref[...],
                    preferred_element_type=jnp.float32)

    ids = sid_ref[...]
    lanes = lax.broadcasted_iota(jnp.int32, (_NB, NUM_SAMPLES), 1)
    oh = (ids == lanes).astype(jnp.float32)
    accs[...] += oh * logit
    accc[...] += oh

    @pl.when(i == pl.num_programs(0) - 1)
    def _fin():
        ssum = accs[...].sum(axis=0, keepdims=True)
        csum = accc[...].sum(axis=0, keepdims=True)
        out_ref[...] = ssum / jnp.maximum(csum, 1.0)


def _tc_call(ge8, ft8, pm8, sid, W8, S8, R8, ONE, wc):
    grid = B // _NB
    return pl.pallas_call(
        _tc_body,
        grid=(grid,),
        in_specs=[
            pl.BlockSpec((_RW, _LW), lambda i: (i, 0)),
            pl.BlockSpec((_RW, _PK * FEAT_DIM), lambda i: (i, 0)),
            pl.BlockSpec((_RW, _PK), lambda i: (i, 0)),
            pl.BlockSpec((_NB, 1), lambda i: (i, 0)),
            pl.BlockSpec((_PK * FEAT_DIM, _LW), lambda i: (0, 0)),
            pl.BlockSpec((_PK, _LW), lambda i: (0, 0)),
            pl.BlockSpec((_LW, D_MODEL), lambda i: (0, 0)),
            pl.BlockSpec((_LW, 1), lambda i: (0, 0)),
            pl.BlockSpec((D_MODEL, 1), lambda i: (0, 0)),
        ],
        out_specs=pl.BlockSpec((1, NUM_SAMPLES), lambda i: (0, 0)),
        out_shape=jax.ShapeDtypeStruct((1, NUM_SAMPLES), jnp.float32),
        scratch_shapes=[
            pltpu.VMEM((_NB, NUM_SAMPLES), jnp.float32),
            pltpu.VMEM((_NB, NUM_SAMPLES), jnp.float32),
        ],
    )(ge8, ft8, pm8, sid, W8, S8, R8, ONE, wc)


def kernel(features, positions, gene_ids, mask, original_sample_indices,
           gene_table, W_feat, w_cls):
    ids1d = gene_ids.reshape(N).astype(jnp.int32)
    ge = _make_sc_gather()(ids1d, gene_table)

    ge8 = ge.reshape(N // _PK, _LW)
    ft8 = features.reshape(N // _PK, _PK * FEAT_DIM)
    pm8 = (positions.astype(jnp.float32) / 1e5
           + (1.0 - mask.astype(jnp.float32)) * MASK_BIG).reshape(N // _PK,
                                                                  _PK)
    sid = original_sample_indices.reshape(B, 1).astype(jnp.int32)
    wc = w_cls.reshape(D_MODEL, 1)

    eye8 = jnp.eye(_PK, dtype=jnp.float32)
    W8 = jnp.kron(eye8, W_feat)
    S8 = jnp.kron(eye8, jnp.ones((1, D_MODEL), jnp.float32))
    R8 = jnp.kron(jnp.ones((_PK, 1), jnp.float32),
                  jnp.eye(D_MODEL, dtype=jnp.float32))
    ONE = jnp.ones((_LW, 1), jnp.float32)

    out = _tc_call(ge8, ft8, pm8, sid, W8, S8, R8, ONE, wc)
    return out.reshape(NUM_SAMPLES)
